# baseline (speedup 1.0000x reference)
import jax
import jax.numpy as jnp
from jax.experimental import pallas as pl
from jax.experimental.pallas import tpu as pltpu

N = 1024
T = 32


def _dummies_body(x_ref, out_ref, tgt_ref):
    t = pl.program_id(0)

    @pl.when(t == 0)
    def _build_targets():
        xm = x_ref[:, 0, :]
        mask = jnp.where(jnp.isnan(xm), 0.0, 1.0)
        ii = jax.lax.broadcasted_iota(jnp.int32, (N, N), 0)
        jj = jax.lax.broadcasted_iota(jnp.int32, (N, N), 1)
        tri = jnp.where(ii <= jj, 1.0, 0.0)
        dest = jax.lax.dot_general(
            mask, tri, (((1,), (0,)), ((), ())),
            preferred_element_type=jnp.float32,
        ) - 1.0
        tgt_ref[...] = jnp.where(mask[:, 1:] > 0.0, dest[:, 1:], -2.0)

    trow = tgt_ref[pl.ds(t, 1), :]
    rows = jax.lax.broadcasted_iota(jnp.int32, (N, 1), 0).astype(jnp.float32)
    out_ref[0, :, :] = jnp.where(trow == rows, 1.0, 0.0)


def _impl(x):
    xs = x.reshape(T, 1, N)
    out = pl.pallas_call(
        _dummies_body,
        grid=(T,),
        in_specs=[pl.BlockSpec((T, 1, N), lambda t: (0, 0, 0))],
        out_specs=pl.BlockSpec((1, N, N - 1), lambda t: (t, 0, 0)),
        out_shape=jax.ShapeDtypeStruct((T, N, N - 1), jnp.float32),
        scratch_shapes=[pltpu.VMEM((T, N - 1), jnp.float32)],
    )(xs)
    return out.reshape(1, T * N, N - 1)


kernel = jax.jit(_impl)


# speedup vs baseline: 1.0112x; 1.0112x over previous
import jax
import jax.numpy as jnp
from jax.experimental import pallas as pl
from jax.experimental.pallas import tpu as pltpu

N = 1024
T = 32


def _dummies_body(x_ref, out_ref, tgt_ref):
    t = pl.program_id(0)

    @pl.when(t == 0)
    def _build_targets():
        xm = x_ref[0]
        mask = jnp.where(jnp.isnan(xm), 0.0, 1.0)
        ii = jax.lax.broadcasted_iota(jnp.int32, (N, N), 0)
        jj = jax.lax.broadcasted_iota(jnp.int32, (N, N), 1)
        tri = jnp.where(ii <= jj, 1.0, 0.0)
        dest = jax.lax.dot_general(
            mask, tri, (((1,), (0,)), ((), ())),
            preferred_element_type=jnp.float32,
        ) - 1.0
        tgt_ref[...] = jnp.where(mask[:, 1:] > 0.0, dest[:, 1:], -2.0)

    trow = tgt_ref[pl.ds(t, 1), :]
    rows = jax.lax.broadcasted_iota(jnp.int32, (N, 1), 0).astype(jnp.float32)
    out_ref[0, :, :] = jnp.where(trow == rows, 1.0, 0.0)


def _impl(x):
    out = pl.pallas_call(
        _dummies_body,
        grid=(T,),
        in_specs=[pl.BlockSpec((1, T, N), lambda t: (0, 0, 0))],
        out_specs=pl.BlockSpec((1, N, N - 1), lambda t: (t, 0, 0)),
        out_shape=jax.ShapeDtypeStruct((T, N, N - 1), jnp.float32),
        scratch_shapes=[pltpu.VMEM((T, N - 1), jnp.float32)],
    )(x)
    return out.reshape(1, T * N, N - 1)


kernel = jax.jit(_impl)
